# baseline (speedup 1.0000x reference)
import functools

import jax
import jax.numpy as jnp
from jax import lax
from jax.experimental import pallas as pl
from jax.experimental.pallas import tpu as pltpu
from jax.experimental.pallas import tpu_sc as plsc

B = 4
N = 2048
DIM = 256
K = 20
ROWS = 256

SC_CORES = 2
SC_SUBCORES = 16
SC_WORKERS = SC_CORES * SC_SUBCORES
CHUNK = 128


def _topk_body(batch, rows_ref, pc_ref, sqr_ref, sqc_ref, idx_ref):
    rows = rows_ref[0]
    pc = pc_ref[0]
    inner = lax.dot_general(
        rows, pc, (((1,), (1,)), ((), ())),
        preferred_element_type=jnp.float32,
        precision=lax.Precision.DEFAULT,
    )
    sq_r = sqr_ref[0]
    sq_c = sqc_ref[0]
    d = (sq_r + (-2.0 * inner)) + sq_c

    iota_f = lax.broadcasted_iota(jnp.int32, (ROWS, N), 1).astype(jnp.float32)
    cols = []
    for _ in range(K):
        m = jnp.min(d, axis=1, keepdims=True)
        cand = jnp.where(d == m, iota_f, jnp.float32(N))
        im_f = jnp.min(cand, axis=1, keepdims=True)
        cols.append(im_f)
        d = jnp.where(iota_f == im_f, jnp.float32(jnp.inf), d)
    idx = jnp.concatenate(cols, axis=1).astype(jnp.int32)
    idx_ref[0] = idx + batch * N


def _tc_topk(point_cloud, square, b):
    return pl.pallas_call(
        functools.partial(_topk_body, b),
        grid=(1, N // ROWS),
        in_specs=[
            pl.BlockSpec((1, ROWS, DIM), lambda b, r: (b, r, 0)),
            pl.BlockSpec((1, N, DIM), lambda b, r: (b, 0, 0)),
            pl.BlockSpec((1, ROWS, 1), lambda b, r: (b, r, 0)),
            pl.BlockSpec((1, 1, N), lambda b, r: (b, 0, 0)),
        ],
        out_specs=pl.BlockSpec((1, ROWS, K), lambda b, r: (b, r, 0)),
        out_shape=jax.ShapeDtypeStruct((1, N, K), jnp.int32),
    )(point_cloud, point_cloud, square, square.reshape(1, 1, N))


def _sc_gather(table, idx_flat):
    total = idx_flat.shape[0]
    per_w = total // SC_WORKERS
    n_pairs = per_w // (2 * CHUNK)
    mesh = plsc.VectorSubcoreMesh(core_axis_name="c", subcore_axis_name="s")

    @functools.partial(
        pl.kernel,
        mesh=mesh,
        out_type=jax.ShapeDtypeStruct((total, DIM), jnp.float32),
        compiler_params=pltpu.CompilerParams(use_tc_tiling_on_sc=True),
        scratch_types=[
            pltpu.VMEM((per_w,), jnp.int32),
            pltpu.VMEM((CHUNK, DIM), jnp.float32),
            pltpu.VMEM((CHUNK, DIM), jnp.float32),
            pltpu.SemaphoreType.DMA,
            pltpu.SemaphoreType.DMA,
            pltpu.SemaphoreType.DMA,
            pltpu.SemaphoreType.DMA,
        ],
    )
    def gather_kernel(table_hbm, idx_hbm, out_hbm, idx_v, rows0, rows1,
                      g0, g1, s0, s1):
        wid = lax.axis_index("s") * SC_CORES + lax.axis_index("c")
        base = wid * per_w
        pltpu.sync_copy(idx_hbm.at[pl.ds(base, per_w)], idx_v)

        def gather(c, buf, sem):
            return pltpu.make_async_copy(
                table_hbm.at[idx_v.at[pl.ds(c * CHUNK, CHUNK)]], buf, sem)

        def store(c, buf, sem):
            return pltpu.make_async_copy(
                buf, out_hbm.at[pl.ds(base + c * CHUNK, CHUNK)], sem)

        gather(0, rows0, g0).start()

        @pl.loop(0, n_pairs)
        def _(i):
            c0 = 2 * i
            c1 = c0 + 1
            gather(c0, rows0, g0).wait()

            @pl.when(i > 0)
            def _():
                store(c1 - 2, rows1, s1).wait()

            gather(c1, rows1, g1).start()
            store(c0, rows0, s0).start()
            gather(c1, rows1, g1).wait()
            store(c0, rows0, s0).wait()

            @pl.when(i < n_pairs - 1)
            def _():
                gather(c0 + 2, rows0, g0).start()

            store(c1, rows1, s1).start()

        store(per_w // CHUNK - 1, rows1, s1).wait()

    return gather_kernel(table, idx_flat)


def kernel(point_cloud):
    square = jnp.sum(jnp.square(point_cloud), axis=-1, keepdims=True)
    table = point_cloud.reshape(B * N, DIM)
    outs = []
    for b in range(B):
        nn_idx = _tc_topk(point_cloud[b:b + 1], square[b:b + 1], b)
        rows = _sc_gather(table, nn_idx.reshape(-1))
        outs.append(rows.reshape(N, K, DIM))
    return jnp.stack(outs, axis=0)


# speedup vs baseline: 1.0646x; 1.0646x over previous
import functools

import jax
import jax.numpy as jnp
from jax import lax
from jax.experimental import pallas as pl
from jax.experimental.pallas import tpu as pltpu
from jax.experimental.pallas import tpu_sc as plsc

B = 4
N = 2048
DIM = 256
K = 20
ROWS = 512

SC_CORES = 2
SC_SUBCORES = 16
SC_WORKERS = SC_CORES * SC_SUBCORES
CHUNK = 128


def _topk_body(rows_ref, pc_ref, sqr_ref, sqc_ref, idx_ref):
    rows = rows_ref[0]
    pc = pc_ref[0]
    inner = lax.dot_general(
        rows, pc, (((1,), (1,)), ((), ())),
        preferred_element_type=jnp.float32,
        precision=lax.Precision.DEFAULT,
    )
    sq_r = sqr_ref[0]
    sq_c = sqc_ref[0]
    d = (sq_r + (-2.0 * inner)) + sq_c

    iota_f = lax.broadcasted_iota(jnp.int32, (ROWS, N), 1).astype(jnp.float32)
    cols = []
    for _ in range(K):
        m = jnp.min(d, axis=1, keepdims=True)
        cand = jnp.where(d == m, iota_f, jnp.float32(N))
        im_f = jnp.min(cand, axis=1, keepdims=True)
        cols.append(im_f)
        d = jnp.where(iota_f == im_f, jnp.float32(jnp.inf), d)
    off = pl.program_id(0) * N
    idx = jnp.concatenate(cols, axis=1).astype(jnp.int32)
    idx_ref[0] = idx + off


def _tc_topk(point_cloud, square):
    return pl.pallas_call(
        _topk_body,
        grid=(B, N // ROWS),
        in_specs=[
            pl.BlockSpec((1, ROWS, DIM), lambda b, r: (b, r, 0)),
            pl.BlockSpec((1, N, DIM), lambda b, r: (b, 0, 0)),
            pl.BlockSpec((1, ROWS, 1), lambda b, r: (b, r, 0)),
            pl.BlockSpec((1, 1, N), lambda b, r: (b, 0, 0)),
        ],
        out_specs=pl.BlockSpec((1, ROWS, K), lambda b, r: (b, r, 0)),
        out_shape=jax.ShapeDtypeStruct((B, N, K), jnp.int32),
    )(point_cloud, point_cloud, square, square.reshape(B, 1, N))


def _sc_gather(table, idx_flat):
    total = idx_flat.shape[0]
    per_w = total // SC_WORKERS
    n_pairs = per_w // (2 * CHUNK)
    mesh = plsc.VectorSubcoreMesh(core_axis_name="c", subcore_axis_name="s")

    @functools.partial(
        pl.kernel,
        mesh=mesh,
        out_type=jax.ShapeDtypeStruct((total, DIM), jnp.float32),
        compiler_params=pltpu.CompilerParams(use_tc_tiling_on_sc=True),
        scratch_types=[
            pltpu.VMEM((per_w,), jnp.int32),
            pltpu.VMEM((CHUNK, DIM), jnp.float32),
            pltpu.VMEM((CHUNK, DIM), jnp.float32),
            pltpu.SemaphoreType.DMA,
            pltpu.SemaphoreType.DMA,
            pltpu.SemaphoreType.DMA,
            pltpu.SemaphoreType.DMA,
        ],
    )
    def gather_kernel(table_hbm, idx_hbm, out_hbm, idx_v, rows0, rows1,
                      g0, g1, s0, s1):
        wid = lax.axis_index("s") * SC_CORES + lax.axis_index("c")
        base = wid * per_w
        pltpu.sync_copy(idx_hbm.at[pl.ds(base, per_w)], idx_v)

        def gather(c, buf, sem):
            return pltpu.make_async_copy(
                table_hbm.at[idx_v.at[pl.ds(c * CHUNK, CHUNK)]], buf, sem)

        def store(c, buf, sem):
            return pltpu.make_async_copy(
                buf, out_hbm.at[pl.ds(base + c * CHUNK, CHUNK)], sem)

        gather(0, rows0, g0).start()

        @pl.loop(0, n_pairs)
        def _(i):
            c0 = 2 * i
            c1 = c0 + 1
            gather(c0, rows0, g0).wait()

            @pl.when(i > 0)
            def _():
                store(c1 - 2, rows1, s1).wait()

            gather(c1, rows1, g1).start()
            store(c0, rows0, s0).start()
            gather(c1, rows1, g1).wait()
            store(c0, rows0, s0).wait()

            @pl.when(i < n_pairs - 1)
            def _():
                gather(c0 + 2, rows0, g0).start()

            store(c1, rows1, s1).start()

        store(per_w // CHUNK - 1, rows1, s1).wait()

    return gather_kernel(table, idx_flat)


def kernel(point_cloud):
    square = jnp.sum(jnp.square(point_cloud), axis=-1, keepdims=True)
    nn_idx = _tc_topk(point_cloud, square)
    rows = _sc_gather(point_cloud.reshape(B * N, DIM), nn_idx.reshape(-1))
    return rows.reshape(B, N, K, DIM)
